# baseline (speedup 1.0000x reference)
import jax
import jax.numpy as jnp
from jax.experimental import pallas as pl
from jax.experimental.pallas import tpu as pltpu

RING = 4
LOOK = 3


def _moe_kernel(x_ref, wg_ref, wn_ref, eps_ref, b1_ref, b2_ref,
                w1_hbm, w2_hbm, out_ref,
                hpart_ref, w_ref, buf1_ref, buf2_ref, sem1, sem2):
    n_chunks = w1_hbm.shape[0]
    d_half = w1_hbm.shape[1]
    d_hid = w1_hbm.shape[2]
    h_half = w2_hbm.shape[1]
    n_exp = wg_ref.shape[1]

    def cp1(i, slot):
        return pltpu.make_async_copy(w1_hbm.at[pl.ds(i, 1)],
                                     buf1_ref.at[pl.ds(slot, 1)],
                                     sem1.at[slot])

    def cp2(i, slot):
        return pltpu.make_async_copy(w2_hbm.at[pl.ds(i, 1)],
                                     buf2_ref.at[pl.ds(slot, 1)],
                                     sem2.at[slot])

    for k in range(LOOK):
        cp1(k, k).start()
        cp2(k, k).start()

    xv = x_ref[...]
    g = jnp.dot(xv, wg_ref[...], preferred_element_type=jnp.float32)
    n = jnp.dot(xv, wn_ref[...], preferred_element_type=jnp.float32)
    logits = g + jax.nn.softplus(n) * eps_ref[...]
    lane = jax.lax.broadcasted_iota(jnp.int32, logits.shape, 1)
    i1 = jnp.argmax(logits, axis=1)[:, None]
    v1 = jnp.max(logits, axis=1)[:, None]
    oh1 = lane == i1
    masked = jnp.where(oh1, -jnp.inf, logits)
    i2 = jnp.argmax(masked, axis=1)[:, None]
    v2 = jnp.max(masked, axis=1)[:, None]
    oh2 = lane == i2
    e2 = jnp.exp(v2 - v1)
    denom = 1.0 + e2
    w_ref[...] = jnp.where(oh1, 1.0 / denom, jnp.where(oh2, e2 / denom, 0.0))
    out_ref[...] = jnp.zeros_like(out_ref)

    xa = xv[:, :d_half]
    xb = xv[:, d_half:]

    def body(i, carry):
        e = i // 2
        s = jax.lax.rem(i, RING)
        cp1(i, s).wait()
        cp2(i, s).wait()

        @pl.when(i % 2 == 0)
        def _first_half():
            hpart_ref[...] = jnp.dot(xa, buf1_ref[pl.ds(s, 1)][0],
                                     preferred_element_type=jnp.float32)

        @pl.when(i % 2 == 1)
        def _second_half():
            sp = jax.lax.rem(i - 1, RING)
            h = (hpart_ref[...]
                 + jnp.dot(xb, buf1_ref[pl.ds(s, 1)][0],
                           preferred_element_type=jnp.float32)
                 + b1_ref[pl.ds(e, 1)][0])
            rh = jnp.maximum(h, 0.0)
            acc = (jnp.dot(rh[:, :h_half], buf2_ref[pl.ds(sp, 1)][0],
                           preferred_element_type=jnp.float32)
                   + jnp.dot(rh[:, h_half:], buf2_ref[pl.ds(s, 1)][0],
                             preferred_element_type=jnp.float32))
            lane2 = jax.lax.broadcasted_iota(
                jnp.int32, (out_ref.shape[0], n_exp), 1)
            we = jnp.sum(jnp.where(lane2 == e, w_ref[...], 0.0), axis=1,
                         keepdims=True)
            out_ref[...] += we * (acc + b2_ref[pl.ds(e, 1)][0])

        @pl.when(i + LOOK < n_chunks)
        def _refill():
            ns = jax.lax.rem(i + LOOK, RING)
            cp1(i + LOOK, ns).start()
            cp2(i + LOOK, ns).start()

        return carry

    jax.lax.fori_loop(0, n_chunks, body, 0)


def kernel(x, Wg, Wnoise, W1, b1, W2, b2):
    b, c, d = x.shape
    n_exp, _, d_hid = W1.shape
    t = b * c
    x2 = x.reshape(t, d)
    eps = jax.random.normal(jax.random.key(42), (b, c, n_exp),
                            dtype=x.dtype).reshape(t, n_exp)
    nc = 2 * n_exp
    w1r = W1.reshape(nc, d // 2, d_hid)
    w2r = W2.reshape(nc, d_hid // 2, d)
    out = pl.pallas_call(
        _moe_kernel,
        in_specs=[
            pl.BlockSpec((t, d), lambda: (0, 0)),
            pl.BlockSpec((d, n_exp), lambda: (0, 0)),
            pl.BlockSpec((d, n_exp), lambda: (0, 0)),
            pl.BlockSpec((t, n_exp), lambda: (0, 0)),
            pl.BlockSpec((n_exp, 1, d_hid), lambda: (0, 0, 0)),
            pl.BlockSpec((n_exp, 1, d), lambda: (0, 0, 0)),
            pl.BlockSpec(memory_space=pltpu.MemorySpace.HBM),
            pl.BlockSpec(memory_space=pltpu.MemorySpace.HBM),
        ],
        out_specs=pl.BlockSpec((t, d), lambda: (0, 0)),
        out_shape=jax.ShapeDtypeStruct((t, d), x.dtype),
        scratch_shapes=[
            pltpu.VMEM((t, d_hid), jnp.float32),
            pltpu.VMEM((t, n_exp), jnp.float32),
            pltpu.VMEM((RING, d // 2, d_hid), jnp.float32),
            pltpu.VMEM((RING, d_hid // 2, d), jnp.float32),
            pltpu.SemaphoreType.DMA((RING,)),
            pltpu.SemaphoreType.DMA((RING,)),
        ],
    )(x2, Wg.T, Wnoise.T, eps, b1[:, None, :], b2[:, None, :], w1r, w2r)
    return out.reshape(b, c, d)
